# Initial kernel scaffold; baseline (speedup 1.0000x reference)
import jax
import jax.numpy as jnp
from jax.experimental import pallas as pl


def kernel(input, adj, W, a):
    raise NotImplementedError("write your pallas kernel here")


# speedup vs baseline: 4786.8939x; 4786.8939x over previous
import functools

import jax
import jax.numpy as jnp
from jax.experimental import pallas as pl
from jax.experimental.pallas import tpu as pltpu

_BLOCK_ROWS = 256


def _gat_body(x_ref, adj_ref, w_ref, a_ref, out_ref, wh_ref, t_ref, *, fout):
    i = pl.program_id(0)

    @pl.when(i == 0)
    def _init():
        wh = jnp.dot(x_ref[...], w_ref[...], preferred_element_type=jnp.float32)
        wh_ref[...] = wh
        a2 = a_ref[:, fout:]
        t_ref[...] = jax.lax.dot_general(
            a2, wh, (((1,), (1,)), ((), ())), preferred_element_type=jnp.float32
        )

    br = adj_ref.shape[0]
    wh_blk = wh_ref[pl.ds(i * br, br), :]
    a1 = a_ref[:, :fout]
    s = jax.lax.dot_general(
        wh_blk, a1, (((1,), (1,)), ((), ())), preferred_element_type=jnp.float32
    )
    e = s + t_ref[...]
    e = jnp.where(e >= 0, e, 0.2 * e)
    mask = adj_ref[...] != 0
    m = jnp.max(jnp.where(mask, e, -jnp.inf), axis=1, keepdims=True)
    p = jnp.where(mask, jnp.exp(e - m), 0.0)
    l = jnp.sum(p, axis=1, keepdims=True)
    acc = jnp.dot(p, wh_ref[...], preferred_element_type=jnp.float32)
    h = acc / l
    out_ref[...] = jnp.where(h > 0, h, jnp.exp(h) - 1.0)


def kernel(input, adj, W, a):
    n, fin = input.shape
    fout = W.shape[1]
    br = _BLOCK_ROWS
    grid = (n // br,)
    return pl.pallas_call(
        functools.partial(_gat_body, fout=fout),
        grid=grid,
        in_specs=[
            pl.BlockSpec((n, fin), lambda i: (0, 0)),
            pl.BlockSpec((br, n), lambda i: (i, 0)),
            pl.BlockSpec((fin, fout), lambda i: (0, 0)),
            pl.BlockSpec((1, 2 * fout), lambda i: (0, 0)),
        ],
        out_specs=pl.BlockSpec((br, fout), lambda i: (i, 0)),
        out_shape=jax.ShapeDtypeStruct((n, fout), jnp.float32),
        scratch_shapes=[
            pltpu.VMEM((n, fout), jnp.float32),
            pltpu.VMEM((1, n), jnp.float32),
        ],
    )(input, adj, W, a)


# speedup vs baseline: 5313.4671x; 1.1100x over previous
import functools

import jax
import jax.numpy as jnp
from jax.experimental import pallas as pl
from jax.experimental.pallas import tpu as pltpu

_BLOCK_ROWS = 256


def _gat_body(x_ref, adj_ref, w_ref, a_ref, out_ref, wh_ref, t_ref, whx_ref,
              *, fout):
    i = pl.program_id(0)
    n = x_ref.shape[0]

    @pl.when(i == 0)
    def _init():
        wh = jnp.dot(x_ref[...], w_ref[...], preferred_element_type=jnp.float32)
        wh_ref[...] = wh
        a2 = a_ref[:, fout:]
        t_ref[...] = jax.lax.dot_general(
            a2, wh, (((1,), (1,)), ((), ())), preferred_element_type=jnp.float32
        )
        ext = jnp.concatenate(
            [wh, jnp.ones((n, 1), jnp.float32),
             jnp.zeros((n, 16 - 1), jnp.float32)], axis=1)
        whx_ref[...] = ext.astype(jnp.bfloat16)

    br = adj_ref.shape[0]
    wh_blk = wh_ref[pl.ds(i * br, br), :]
    a1 = a_ref[:, :fout]
    s = jax.lax.dot_general(
        wh_blk, a1, (((1,), (1,)), ((), ())), preferred_element_type=jnp.float32
    )
    t = t_ref[...]
    m = s + jnp.max(t)
    m = jnp.maximum(m, 0.2 * m)
    z = s + t
    lz = jnp.maximum(z, 0.2 * z)
    p = jnp.exp(lz - m) * adj_ref[...].astype(jnp.float32)
    r = jnp.dot(p.astype(jnp.bfloat16), whx_ref[...],
                preferred_element_type=jnp.float32)
    acc = r[:, :fout]
    l = r[:, fout:fout + 1]
    h = acc / l
    out_ref[...] = jnp.where(h > 0, h, jnp.exp(h) - 1.0)


def kernel(input, adj, W, a):
    n, fin = input.shape
    fout = W.shape[1]
    br = _BLOCK_ROWS
    grid = (n // br,)
    return pl.pallas_call(
        functools.partial(_gat_body, fout=fout),
        grid=grid,
        in_specs=[
            pl.BlockSpec((n, fin), lambda i: (0, 0)),
            pl.BlockSpec((br, n), lambda i: (i, 0)),
            pl.BlockSpec((fin, fout), lambda i: (0, 0)),
            pl.BlockSpec((1, 2 * fout), lambda i: (0, 0)),
        ],
        out_specs=pl.BlockSpec((br, fout), lambda i: (i, 0)),
        out_shape=jax.ShapeDtypeStruct((n, fout), jnp.float32),
        scratch_shapes=[
            pltpu.VMEM((n, fout), jnp.float32),
            pltpu.VMEM((1, n), jnp.float32),
            pltpu.VMEM((n, fout + 16), jnp.bfloat16),
        ],
    )(input, adj, W, a)


# speedup vs baseline: 5484.2775x; 1.0321x over previous
import functools

import jax
import jax.numpy as jnp
from jax.experimental import pallas as pl
from jax.experimental.pallas import tpu as pltpu

_BLOCK_ROWS = 256


def _gat_body(x_ref, adj_ref, w_ref, a_ref, out_ref, wh_ref, t_ref, whx_ref,
              *, fout):
    i = pl.program_id(0)
    n = x_ref.shape[0]

    @pl.when(i == 0)
    def _init():
        wh = jnp.dot(x_ref[...], w_ref[...], preferred_element_type=jnp.float32)
        wh_ref[...] = wh
        a2 = a_ref[:, fout:]
        t_ref[...] = jax.lax.dot_general(
            a2, wh, (((1,), (1,)), ((), ())), preferred_element_type=jnp.float32
        )
        ext = jnp.concatenate(
            [wh, jnp.ones((n, 1), jnp.float32),
             jnp.zeros((n, 16 - 1), jnp.float32)], axis=1)
        whx_ref[...] = ext.astype(jnp.bfloat16)

    br = adj_ref.shape[0]
    wh_blk = wh_ref[pl.ds(i * br, br), :]
    a1 = a_ref[:, :fout]
    s = jax.lax.dot_general(
        wh_blk, a1, (((1,), (1,)), ((), ())), preferred_element_type=jnp.float32
    )
    c = jnp.float32(1.4426950408889634)
    sp = s * c
    tp = t_ref[...] * c
    t2 = 0.2 * tp
    mx = sp + jnp.max(tp)
    mp = jnp.maximum(mx, 0.2 * mx)
    u = sp - mp
    v = 0.2 * sp - mp
    arg = jnp.maximum(u + tp, v + t2)
    p = jnp.exp2(arg) * adj_ref[...].astype(jnp.float32)
    r = jnp.dot(p.astype(jnp.bfloat16), whx_ref[...],
                preferred_element_type=jnp.float32)
    acc = r[:, :fout]
    l = r[:, fout:fout + 1]
    h = acc / l
    out_ref[...] = jnp.where(h > 0, h, jnp.exp(h) - 1.0)


def kernel(input, adj, W, a):
    n, fin = input.shape
    fout = W.shape[1]
    br = _BLOCK_ROWS
    grid = (n // br,)
    return pl.pallas_call(
        functools.partial(_gat_body, fout=fout),
        grid=grid,
        in_specs=[
            pl.BlockSpec((n, fin), lambda i: (0, 0)),
            pl.BlockSpec((br, n), lambda i: (i, 0)),
            pl.BlockSpec((fin, fout), lambda i: (0, 0)),
            pl.BlockSpec((1, 2 * fout), lambda i: (0, 0)),
        ],
        out_specs=pl.BlockSpec((br, fout), lambda i: (i, 0)),
        out_shape=jax.ShapeDtypeStruct((n, fout), jnp.float32),
        scratch_shapes=[
            pltpu.VMEM((n, fout), jnp.float32),
            pltpu.VMEM((1, n), jnp.float32),
            pltpu.VMEM((n, fout + 16), jnp.bfloat16),
        ],
    )(input, adj, W, a)


# speedup vs baseline: 5716.8385x; 1.0424x over previous
import functools

import jax
import jax.numpy as jnp
from jax.experimental import pallas as pl
from jax.experimental.pallas import tpu as pltpu

_BLOCK_ROWS = 256


def _gat_body(x_ref, adj_ref, w_ref, a_ref, out_ref, wh_ref, tp_ref, t2_ref,
              whx_ref, *, fout):
    i = pl.program_id(0)
    n = x_ref.shape[0]
    c = jnp.float32(1.4426950408889634)

    @pl.when(i == 0)
    def _init():
        wh = jnp.dot(x_ref[...], w_ref[...], preferred_element_type=jnp.float32)
        wh_ref[...] = wh
        a2 = a_ref[:, fout:]
        tp = c * jax.lax.dot_general(
            a2, wh, (((1,), (1,)), ((), ())), preferred_element_type=jnp.float32
        )
        tp_ref[...] = tp.astype(jnp.bfloat16)
        t2_ref[...] = (0.2 * tp).astype(jnp.bfloat16)
        ext = jnp.concatenate(
            [wh, jnp.ones((n, 1), jnp.float32),
             jnp.zeros((n, 16 - 1), jnp.float32)], axis=1)
        whx_ref[...] = ext.astype(jnp.bfloat16)

    br = adj_ref.shape[0]
    wh_blk = wh_ref[pl.ds(i * br, br), :]
    a1 = a_ref[:, :fout]
    s = jax.lax.dot_general(
        wh_blk, a1, (((1,), (1,)), ((), ())), preferred_element_type=jnp.float32
    )
    sp = c * s
    tmax = jnp.max(tp_ref[...].astype(jnp.float32))
    mx = sp + tmax
    mp = jnp.maximum(mx, 0.2 * mx)
    u = (sp - mp).astype(jnp.bfloat16)
    v = (0.2 * sp - mp).astype(jnp.bfloat16)
    arg = jnp.maximum(u + tp_ref[...], v + t2_ref[...])
    p = jnp.exp2(arg) * adj_ref[...].astype(jnp.bfloat16)
    r = jnp.dot(p, whx_ref[...], preferred_element_type=jnp.float32)
    acc = r[:, :fout]
    l = r[:, fout:fout + 1]
    h = acc / l
    out_ref[...] = jnp.where(h > 0, h, jnp.exp(h) - 1.0)


def kernel(input, adj, W, a):
    n, fin = input.shape
    fout = W.shape[1]
    br = _BLOCK_ROWS
    grid = (n // br,)
    return pl.pallas_call(
        functools.partial(_gat_body, fout=fout),
        grid=grid,
        in_specs=[
            pl.BlockSpec((n, fin), lambda i: (0, 0)),
            pl.BlockSpec((br, n), lambda i: (i, 0)),
            pl.BlockSpec((fin, fout), lambda i: (0, 0)),
            pl.BlockSpec((1, 2 * fout), lambda i: (0, 0)),
        ],
        out_specs=pl.BlockSpec((br, fout), lambda i: (i, 0)),
        out_shape=jax.ShapeDtypeStruct((n, fout), jnp.float32),
        scratch_shapes=[
            pltpu.VMEM((n, fout), jnp.float32),
            pltpu.VMEM((1, n), jnp.bfloat16),
            pltpu.VMEM((1, n), jnp.bfloat16),
            pltpu.VMEM((n, fout + 16), jnp.bfloat16),
        ],
    )(input, adj, W, a)


# speedup vs baseline: 7590.1139x; 1.3277x over previous
import functools

import jax
import jax.numpy as jnp
from jax.experimental import pallas as pl
from jax.experimental.pallas import tpu as pltpu

_BLOCK_ROWS = 256


def _gat_body(x_ref, adj_ref, wt_ref, a_ref, out_ref, wh_ref, tp_ref, t2_ref,
              whx_ref, *, fout):
    i = pl.program_id(0)
    n = x_ref.shape[0]
    c = jnp.float32(1.4426950408889634)

    @pl.when(i == 0)
    def _init():
        wh = jax.lax.dot_general(
            x_ref[...], wt_ref[...], (((1,), (1,)), ((), ())),
            preferred_element_type=jnp.float32)
        wh_ref[...] = wh
        a2 = a_ref[:, fout:]
        tp = c * jax.lax.dot_general(
            a2, wh, (((1,), (1,)), ((), ())), preferred_element_type=jnp.float32
        )
        tp_ref[...] = tp
        t2_ref[...] = 0.2 * tp
        ext = jnp.concatenate(
            [wh, jnp.ones((n, 1), jnp.float32),
             jnp.zeros((n, 16 - 1), jnp.float32)], axis=1)
        whx_ref[...] = ext.astype(jnp.bfloat16)

    br = adj_ref.shape[0]
    wh_blk = wh_ref[pl.ds(i * br, br), :]
    a1 = a_ref[:, :fout]
    s = jax.lax.dot_general(
        wh_blk, a1, (((1,), (1,)), ((), ())), preferred_element_type=jnp.float32
    )
    sp = c * s
    tmax = jnp.max(tp_ref[...])
    mx = sp + tmax
    mp = jnp.maximum(mx, 0.2 * mx)
    u = sp - mp
    v = 0.2 * sp - mp
    arg = jnp.maximum(u + tp_ref[...], v + t2_ref[...])
    p = (jnp.exp2(arg) * adj_ref[...].astype(jnp.float32)).astype(jnp.bfloat16)
    rt = jax.lax.dot_general(
        whx_ref[...], p, (((0,), (1,)), ((), ())),
        preferred_element_type=jnp.float32)
    acc = rt[:fout, :]
    l = rt[fout:fout + 1, :]
    h = acc / l
    out_ref[...] = jnp.where(h > 0, h, jnp.exp(h) - 1.0)


def kernel(input, adj, W, a):
    n, fin = input.shape
    fout = W.shape[1]
    br = _BLOCK_ROWS
    grid = (n // br,)
    out_t = pl.pallas_call(
        functools.partial(_gat_body, fout=fout),
        grid=grid,
        in_specs=[
            pl.BlockSpec((n, fin), lambda i: (0, 0)),
            pl.BlockSpec((br, n), lambda i: (i, 0)),
            pl.BlockSpec((fout, fin), lambda i: (0, 0)),
            pl.BlockSpec((1, 2 * fout), lambda i: (0, 0)),
        ],
        out_specs=pl.BlockSpec((fout, br), lambda i: (0, i)),
        out_shape=jax.ShapeDtypeStruct((fout, n), jnp.float32),
        scratch_shapes=[
            pltpu.VMEM((n, fout), jnp.float32),
            pltpu.VMEM((1, n), jnp.float32),
            pltpu.VMEM((1, n), jnp.float32),
            pltpu.VMEM((n, fout + 16), jnp.bfloat16),
        ],
    )(input, adj, W.T, a)
    return out_t.T
